# baseline (speedup 1.0000x reference)
import functools

import jax
import jax.numpy as jnp
from jax import lax
from jax.experimental import pallas as pl
from jax.experimental.pallas import tpu as pltpu
from jax.experimental.pallas import tpu_sc as plsc

BATCH = 4096
EMB = 32
NC, NS = 2, 16
NW = NC * NS
BPW = BATCH // NW
NJ = 8
JBLK = BATCH // NJ


def _sc_gather(user_ids, item_ids, U, Q, A, B):
    mesh = plsc.VectorSubcoreMesh(core_axis_name="c", subcore_axis_name="s")

    @functools.partial(
        pl.kernel,
        mesh=mesh,
        compiler_params=pltpu.CompilerParams(use_tc_tiling_on_sc=False),
        out_type=[
            jax.ShapeDtypeStruct((BATCH, EMB), jnp.float32),
            jax.ShapeDtypeStruct((BATCH, EMB), jnp.float32),
            jax.ShapeDtypeStruct((BATCH,), jnp.float32),
            jax.ShapeDtypeStruct((BATCH,), jnp.float32),
        ],
        scratch_types=[
            pltpu.VMEM((BPW,), jnp.int32),
            pltpu.VMEM((BPW,), jnp.int32),
            pltpu.VMEM((BPW, EMB), jnp.float32),
            pltpu.VMEM((BPW, EMB), jnp.float32),
            pltpu.VMEM((BPW,), jnp.float32),
            pltpu.VMEM((BPW,), jnp.float32),
            pltpu.SemaphoreType.DMA,
            pltpu.SemaphoreType.DMA,
            pltpu.SemaphoreType.DMA,
            pltpu.SemaphoreType.DMA,
        ],
    )
    def sc_kernel(uids_hbm, iids_hbm, u_hbm, q_hbm, a_hbm, b_hbm,
                  uv_hbm, qv_hbm, ag_hbm, bg_hbm,
                  uidx_v, iidx_v, uv_v, qv_v, a_v, b_v,
                  sem_u, sem_q, sem_a, sem_b):
        wid = lax.axis_index("s") * NC + lax.axis_index("c")
        base = wid * BPW
        pltpu.sync_copy(uids_hbm.at[pl.ds(base, BPW)], uidx_v)
        pltpu.sync_copy(iids_hbm.at[pl.ds(base, BPW)], iidx_v)
        cu = pltpu.async_copy(u_hbm.at[uidx_v], uv_v, sem_u)
        cq = pltpu.async_copy(q_hbm.at[iidx_v], qv_v, sem_q)
        ca = pltpu.async_copy(a_hbm.at[uidx_v], a_v, sem_a)
        cb = pltpu.async_copy(b_hbm.at[iidx_v], b_v, sem_b)
        cu.wait()
        cq.wait()
        ca.wait()
        cb.wait()
        pltpu.sync_copy(uv_v, uv_hbm.at[pl.ds(base, BPW)])
        pltpu.sync_copy(qv_v, qv_hbm.at[pl.ds(base, BPW)])
        pltpu.sync_copy(a_v, ag_hbm.at[pl.ds(base, BPW)])
        pltpu.sync_copy(b_v, bg_hbm.at[pl.ds(base, BPW)])

    return sc_kernel(user_ids, item_ids, U, Q, A, B)


def _tc_body(uvj_ref, qvj_ref, ag_ref, bg_ref, uv_ref, qv_ref,
             w1_ref, b1_ref, w2_ref, b2_ref,
             pred_ref, score_ref):
    uqj = uvj_ref[...] * qvj_ref[...]
    rs_row = jnp.sum(uqj, axis=1)[None, :]
    pred_ref[...] = (ag_ref[...] + bg_ref[...]) + rs_row

    @pl.when(pl.program_id(0) == 0)
    def _():
        uv = uv_ref[...]
        qv = qv_ref[...]
        uq = uv * qv
        h = jnp.dot(uv, w1_ref[0:EMB, :], preferred_element_type=jnp.float32)
        h = h + jnp.dot(qv, w1_ref[EMB:2 * EMB, :],
                        preferred_element_type=jnp.float32)
        h = h + jnp.dot(uq, w1_ref[2 * EMB:3 * EMB, :],
                        preferred_element_type=jnp.float32)
        h = jnp.maximum(h + b1_ref[...], 0.0)
        score_ref[...] = (jnp.dot(h, w2_ref[...],
                                  preferred_element_type=jnp.float32)
                          + b2_ref[...])


def _tc_stage(uv, qv, ag, bg, W1, b1r, W2, b2r, interpret=False):
    return pl.pallas_call(
        _tc_body,
        grid=(NJ,),
        in_specs=[
            pl.BlockSpec((JBLK, EMB), lambda j: (j, 0)),
            pl.BlockSpec((JBLK, EMB), lambda j: (j, 0)),
            pl.BlockSpec((BATCH, 1), lambda j: (0, 0)),
            pl.BlockSpec((BATCH, 1), lambda j: (0, 0)),
            pl.BlockSpec((BATCH, EMB), lambda j: (0, 0)),
            pl.BlockSpec((BATCH, EMB), lambda j: (0, 0)),
            pl.BlockSpec((3 * EMB, 64), lambda j: (0, 0)),
            pl.BlockSpec((1, 64), lambda j: (0, 0)),
            pl.BlockSpec((64, 1), lambda j: (0, 0)),
            pl.BlockSpec((1, 1), lambda j: (0, 0)),
        ],
        out_specs=[
            pl.BlockSpec((BATCH, JBLK), lambda j: (0, j)),
            pl.BlockSpec((BATCH, 1), lambda j: (0, 0)),
        ],
        out_shape=[
            jax.ShapeDtypeStruct((BATCH, BATCH), jnp.float32),
            jax.ShapeDtypeStruct((BATCH, 1), jnp.float32),
        ],
        interpret=interpret,
    )(uv, qv, ag, bg, uv, qv, W1, b1r, W2, b2r)


def kernel(user_ids, item_ids, U, Q, A, B, W1, b1, W2, b2):
    user_ids = user_ids.astype(jnp.int32)
    item_ids = item_ids.astype(jnp.int32)
    uv, qv, ag, bg = _sc_gather(user_ids, item_ids, U, Q,
                                A.reshape(-1), B.reshape(-1))
    ag = ag.reshape(BATCH, 1)
    bg = bg.reshape(BATCH, 1)
    pred, score = _tc_stage(
        uv, qv, ag, bg, W1,
        b1.reshape(1, 64),
        W2,
        b2.reshape(1, 1),
    )
    return (pred, score)
